# baseline (speedup 1.0000x reference)
import functools

import jax
import jax.numpy as jnp
from jax import lax
from jax.experimental import pallas as pl


def _scores_body(q_ref, k_ref, out_ref, *, kb, n_valid):
    j = pl.program_id(0)
    qb = q_ref[...]
    qnorm = jnp.sqrt(jnp.sum(qb * qb, axis=1, keepdims=True))
    qn = qb / jnp.clip(qnorm, 1e-12, None)
    kblk = k_ref[...]
    knorm = jnp.sqrt(jnp.sum(kblk * kblk, axis=1, keepdims=True))
    kn = kblk / jnp.clip(knorm, 1e-12, None)
    s = lax.dot_general(qn, kn, (((1,), (1,)), ((), ())),
                        preferred_element_type=jnp.float32)
    col = j * kb + lax.broadcasted_iota(jnp.int32, s.shape, 1)
    out_ref[...] = jnp.where(col < n_valid, s, -1e30)


def _scores(q, keys, kb=2048):
    qn_rows, d = q.shape
    n = keys.shape[0]
    grid = -(-n // kb)
    npad = grid * kb
    body = functools.partial(_scores_body, kb=kb, n_valid=n)
    return pl.pallas_call(
        body,
        grid=(grid,),
        in_specs=[
            pl.BlockSpec((qn_rows, d), lambda j: (0, 0)),
            pl.BlockSpec((kb, d), lambda j: (j, 0)),
        ],
        out_specs=pl.BlockSpec((qn_rows, kb), lambda j: (0, j)),
        out_shape=jax.ShapeDtypeStruct((qn_rows, npad), jnp.float32),
    )(q, keys)


def kernel(q, keys, vals, topk):
    k = min(32, keys.shape[0])
    scores = _scores(q, keys)
    top_vals, idx = lax.top_k(scores, k)
    att = jax.nn.softmax(top_vals, axis=-1)
    v_sel = jnp.take(vals, idx, axis=0)
    v = jnp.sum(att[..., None] * v_sel, axis=1)
    return (v, idx, att)


# speedup vs baseline: 3.2530x; 3.2530x over previous
import functools

import jax
import jax.numpy as jnp
from jax import lax
from jax.experimental import pallas as pl
from jax.experimental.pallas import tpu as pltpu
from jax.experimental.pallas import tpu_sc as plsc

NEG = -3.0e38
K = 32


def _scores_body(q_ref, k_ref, out_ref, *, kb, n_valid):
    j = pl.program_id(0)
    qb = q_ref[...]
    qnorm = jnp.sqrt(jnp.sum(qb * qb, axis=1, keepdims=True))
    qn = qb / jnp.clip(qnorm, 1e-12, None)
    kblk = k_ref[...]
    knorm = jnp.sqrt(jnp.sum(kblk * kblk, axis=1, keepdims=True))
    kn = kblk / jnp.clip(knorm, 1e-12, None)
    s = lax.dot_general(qn, kn, (((1,), (1,)), ((), ())),
                        preferred_element_type=jnp.float32)
    col = j * kb + lax.broadcasted_iota(jnp.int32, s.shape, 1)
    out_ref[...] = jnp.where(col < n_valid, s, -1e30)


def _scores(q, keys, kb=2048):
    nq, d = q.shape
    n = keys.shape[0]
    grid = -(-n // kb)
    npad = grid * kb
    body = functools.partial(_scores_body, kb=kb, n_valid=n)
    return pl.pallas_call(
        body,
        grid=(grid,),
        in_specs=[
            pl.BlockSpec((nq, d), lambda j: (0, 0)),
            pl.BlockSpec((kb, d), lambda j: (j, 0)),
        ],
        out_specs=pl.BlockSpec((nq, kb), lambda j: (0, j)),
        out_shape=jax.ShapeDtypeStruct((nq, npad), jnp.float32),
    )(q, keys)


_GATHER_DNUMS = lax.GatherDimensionNumbers(
    offset_dims=(), collapsed_slice_dims=(0,), start_index_map=(0,))


def _perm(v, idx):
    return lax.gather(v, idx[:, None], _GATHER_DNUMS, (1,),
                      mode=lax.GatherScatterMode.PROMISE_IN_BOUNDS)


def _bfly(v, op):
    lane = lax.iota(jnp.int32, 16)
    for k in (8, 4, 2, 1):
        v = op(v, _perm(v, lane ^ k))
    return v


def _bmax(x):
    return _bfly(x, jnp.maximum)


def _bmin(x):
    return _bfly(x, jnp.minimum)


def _bsum(x):
    return _bfly(x, jnp.add)


def _scal(x):
    return jnp.squeeze(lax.slice(x, (0,), (1,)))


def _topk_combine_sc(scores, vals):
    nq, npad = scores.shape
    nv, d = vals.shape
    info = plsc.get_sparse_core_info()
    ncores, nsub = info.num_cores, info.num_subcores
    nw = ncores * nsub
    qpw = nq // nw
    half = npad // 2
    unroll = 8
    ngroups = half // (16 * unroll)
    assert half % (16 * unroll) == 0
    nd = d // 16

    mesh = plsc.VectorSubcoreMesh(core_axis_name="c", subcore_axis_name="s")

    @functools.partial(
        pl.kernel,
        out_type=[
            jax.ShapeDtypeStruct((nq, d), jnp.float32),
            jax.ShapeDtypeStruct((nq, K), jnp.int32),
            jax.ShapeDtypeStruct((nq, K), jnp.float32),
        ],
        mesh=mesh,
        scratch_types=[
            pltpu.VMEM((2, half), jnp.float32),
            pltpu.VMEM((K,), jnp.float32),
            pltpu.VMEM((K,), jnp.float32),
            pltpu.VMEM((16,), jnp.float32),
            pltpu.VMEM((K,), jnp.int32),
            pltpu.VMEM((K,), jnp.float32),
            pltpu.VMEM((K, d), jnp.float32),
            pltpu.VMEM((d,), jnp.float32),
            pltpu.SemaphoreType.DMA,
            pltpu.SemaphoreType.DMA,
        ],
    )
    def topk_kernel(scores_hbm, vals_hbm, v_out, idx_out, att_out,
                    sbuf, bv_r, bi_r, tau_r, ti_v, att_v, rows_v, vacc_v,
                    sem0, sem1):
        wid = lax.axis_index("s") * ncores + lax.axis_index("c")
        q0 = wid * qpw
        lane = lax.iota(jnp.int32, 16)
        lane_f = lane.astype(jnp.float32)
        negv = jnp.full((16,), NEG, jnp.float32)
        ntasks = qpw * 2

        def reset_state():
            bv_r[pl.ds(0, 16)] = negv
            bv_r[pl.ds(16, 16)] = negv
            bi_r[pl.ds(0, 16)] = lane_f
            bi_r[pl.ds(16, 16)] = lane_f + 16.0
            tau_r[...] = negv

        reset_state()

        pltpu.async_copy(scores_hbm.at[q0, pl.ds(0, half)], sbuf.at[0], sem0)

        def insert(val, gidx):
            tauv = tau_r[...]
            b0 = bv_r[pl.ds(0, 16)]
            b1 = bv_r[pl.ds(16, 16)]
            i0 = bi_r[pl.ds(0, 16)]
            i1 = bi_r[pl.ds(16, 16)]
            ism0 = b0 == tauv
            ism1 = b1 == tauv
            ev = _bmax(jnp.maximum(jnp.where(ism0, i0, -1.0),
                                   jnp.where(ism1, i1, -1.0)))
            r0 = ism0 & (i0 == ev)
            r1 = ism1 & (i1 == ev)
            nb0 = jnp.where(r0, val, b0)
            nb1 = jnp.where(r1, val, b1)
            bv_r[pl.ds(0, 16)] = nb0
            bv_r[pl.ds(16, 16)] = nb1
            bi_r[pl.ds(0, 16)] = jnp.where(r0, gidx, i0)
            bi_r[pl.ds(16, 16)] = jnp.where(r1, gidx, i1)
            tau_r[...] = _bmin(jnp.minimum(nb0, nb1))

        def scan_vreg(h, off, gcol0):
            vec = sbuf[h, pl.ds(off, 16)]
            tauv = tau_r[...]
            mf = jnp.where(vec > tauv, 1.0, 0.0)
            cnum = _scal(_bsum(mf)).astype(jnp.int32)
            col0 = gcol0.astype(jnp.float32)

            def cand_body(i, mcur):
                lf = _bmin(jnp.where(mcur > 0.0, lane_f, 99.0))
                sel = lane_f == lf
                val = _bmax(jnp.where(sel, vec, NEG))
                tauv2 = tau_r[...]
                pred = _scal(jnp.where(val > tauv2, 1.0, 0.0)) > 0.0

                @pl.when(pred)
                def _():
                    insert(val, col0 + lf)

                return jnp.where(sel, 0.0, mcur)

            lax.fori_loop(0, cnum, cand_body, mf)

        def task_body(t, carry):
            qi = q0 + t // 2
            h = t % 2

            @pl.when(t + 1 < ntasks)
            def _start_next():
                tn = t + 1
                pltpu.async_copy(
                    scores_hbm.at[q0 + tn // 2, pl.ds((tn % 2) * half, half)],
                    sbuf.at[tn % 2], sem0)

            pltpu.make_async_copy(
                scores_hbm.at[q0, pl.ds(0, half)], sbuf.at[h], sem0).wait()

            hoff = h * half

            def group_body(g, carry2):
                base = g * (16 * unroll)
                tauv = tau_r[...]
                m = sbuf[h, pl.ds(base, 16)] > tauv
                for j in range(1, unroll):
                    m = m | (sbuf[h, pl.ds(base + j * 16, 16)] > tauv)
                anyhit = _scal(_bmax(jnp.where(m, 1.0, 0.0))) > 0.0

                @pl.when(anyhit)
                def _hit():
                    for j in range(unroll):
                        scan_vreg(h, base + j * 16, hoff + base + j * 16)

                return carry2

            lax.fori_loop(0, ngroups, group_body, 0)

            @pl.when(h == 1)
            def _finalize():
                b0 = bv_r[pl.ds(0, 16)]
                b1 = bv_r[pl.ds(16, 16)]
                i0 = bi_r[pl.ds(0, 16)]
                i1 = bi_r[pl.ds(16, 16)]
                outv = [jnp.zeros((16,), jnp.float32) for _ in range(2)]
                outi = [jnp.zeros((16,), jnp.float32) for _ in range(2)]
                big = jnp.float32(3.0e38)
                for s in range(K):
                    mx = _bmax(jnp.maximum(b0, b1))
                    t0 = b0 == mx
                    t1 = b1 == mx
                    mi = _bmin(jnp.minimum(jnp.where(t0, i0, big),
                                           jnp.where(t1, i1, big)))
                    w = lane == (s % 16)
                    outv[s // 16] = jnp.where(w, mx, outv[s // 16])
                    outi[s // 16] = jnp.where(w, mi, outi[s // 16])
                    rm0 = t0 & (i0 == mi)
                    rm1 = t1 & (i1 == mi)
                    b0 = jnp.where(rm0, NEG, b0)
                    b1 = jnp.where(rm1, NEG, b1)
                mx0 = _bmax(outv[0])
                e0 = jnp.exp(outv[0] - mx0)
                e1 = jnp.exp(outv[1] - mx0)
                ssum = _bsum(e0) + _bsum(e1)
                a0 = e0 / ssum
                a1 = e1 / ssum
                ti_v[pl.ds(0, 16)] = outi[0].astype(jnp.int32)
                ti_v[pl.ds(16, 16)] = outi[1].astype(jnp.int32)
                att_v[pl.ds(0, 16)] = a0
                att_v[pl.ds(16, 16)] = a1
                pltpu.async_copy(vals_hbm.at[ti_v], rows_v, sem1).wait()
                acc = [jnp.zeros((16,), jnp.float32) for _ in range(nd)]
                for jj in range(K):
                    aj = a0 if jj < 16 else a1
                    ww = lane == (jj % 16)
                    ascal = _bmax(jnp.where(ww, aj, -1.0))
                    for dd in range(nd):
                        acc[dd] = acc[dd] + ascal * rows_v[jj, pl.ds(dd * 16, 16)]
                for dd in range(nd):
                    vacc_v[pl.ds(dd * 16, 16)] = acc[dd]
                pltpu.sync_copy(vacc_v, v_out.at[qi])
                pltpu.sync_copy(ti_v, idx_out.at[qi])
                pltpu.sync_copy(att_v, att_out.at[qi])
                reset_state()

            return carry

        lax.fori_loop(0, ntasks, task_body, 0)

    return topk_kernel(scores, vals)


def kernel(q, keys, vals, topk):
    scores = _scores(q, keys)
    v, idx, att = _topk_combine_sc(scores, vals)
    return (v, idx, att)


# speedup vs baseline: 7.3969x; 2.2739x over previous
import functools

import jax
import jax.numpy as jnp
from jax import lax
from jax.experimental import pallas as pl
from jax.experimental.pallas import tpu as pltpu
from jax.experimental.pallas import tpu_sc as plsc

NEG = -3.0e38
K = 32
CHUNK = 128
KB = 2048


def _scores_body(q_ref, k_ref, out_ref, cmax_ref, *, kb, n_valid):
    j = pl.program_id(0)
    qb = q_ref[...]
    qnorm = jnp.sqrt(jnp.sum(qb * qb, axis=1, keepdims=True))
    qn = qb / jnp.clip(qnorm, 1e-12, None)
    kblk = k_ref[...]
    knorm = jnp.sqrt(jnp.sum(kblk * kblk, axis=1, keepdims=True))
    kn = kblk / jnp.clip(knorm, 1e-12, None)
    s = lax.dot_general(qn, kn, (((1,), (1,)), ((), ())),
                        preferred_element_type=jnp.float32)
    col = j * kb + lax.broadcasted_iota(jnp.int32, s.shape, 1)
    s = jnp.where(col < n_valid, s, -1e30)
    out_ref[...] = s
    nq = s.shape[0]
    cmax_ref[...] = jnp.max(s.reshape(nq, kb // CHUNK, CHUNK),
                            axis=2).reshape(1, nq, kb // CHUNK)


def _scores(q, keys, npad, n):
    nq, d = q.shape
    grid = npad // KB
    nchunks = npad // CHUNK
    body = functools.partial(_scores_body, kb=KB, n_valid=n)
    return pl.pallas_call(
        body,
        grid=(grid,),
        in_specs=[
            pl.BlockSpec((nq, d), lambda j: (0, 0)),
            pl.BlockSpec((KB, d), lambda j: (j, 0)),
        ],
        out_specs=[
            pl.BlockSpec((nq, KB), lambda j: (0, j)),
            pl.BlockSpec((1, nq, KB // CHUNK), lambda j: (j, 0, 0)),
        ],
        out_shape=[
            jax.ShapeDtypeStruct((nq, npad), jnp.float32),
            jax.ShapeDtypeStruct((grid, nq, KB // CHUNK), jnp.float32),
        ],
    )(q, keys)


_GATHER_DNUMS = lax.GatherDimensionNumbers(
    offset_dims=(), collapsed_slice_dims=(0,), start_index_map=(0,))


def _perm(v, idx):
    return lax.gather(v, idx[:, None], _GATHER_DNUMS, (1,),
                      mode=lax.GatherScatterMode.PROMISE_IN_BOUNDS)


def _bfly(v, op):
    lane = lax.iota(jnp.int32, 16)
    for k in (8, 4, 2, 1):
        v = op(v, _perm(v, lane ^ k))
    return v


def _bmax(x):
    return _bfly(x, jnp.maximum)


def _bmin(x):
    return _bfly(x, jnp.minimum)


def _bsum(x):
    return _bfly(x, jnp.add)


def _scal(x):
    return jnp.squeeze(lax.slice(x, (0,), (1,)))


def _topk_combine_sc(scores, cmax, vals):
    nq, npad = scores.shape
    nv, d = vals.shape
    info = plsc.get_sparse_core_info()
    ncores, nsub = info.num_cores, info.num_subcores
    nw = ncores * nsub
    qpw = nq // nw
    half = npad // 2
    chalf = half // CHUNK
    nmv = chalf // 16
    assert chalf % 16 == 0
    nd = d // 16

    mesh = plsc.VectorSubcoreMesh(core_axis_name="c", subcore_axis_name="s")

    @functools.partial(
        pl.kernel,
        out_type=[
            jax.ShapeDtypeStruct((nq, d), jnp.float32),
            jax.ShapeDtypeStruct((nq, K), jnp.int32),
            jax.ShapeDtypeStruct((nq, K), jnp.float32),
        ],
        mesh=mesh,
        scratch_types=[
            pltpu.VMEM((2, half), jnp.float32),
            pltpu.VMEM((2, chalf // 16, 16), jnp.float32),
            pltpu.VMEM((K,), jnp.int32),
            pltpu.VMEM((K,), jnp.float32),
            pltpu.VMEM((K, d), jnp.float32),
            pltpu.VMEM((d,), jnp.float32),
            pltpu.SemaphoreType.DMA,
            pltpu.SemaphoreType.DMA,
        ],
    )
    def topk_kernel(scores_hbm, cmax_hbm, vals_hbm, v_out, idx_out, att_out,
                    sbuf, cbuf, ti_v, att_v, rows_v, vacc_v, sem0, sem1):
        wid = lax.axis_index("s") * ncores + lax.axis_index("c")
        q0 = wid * qpw
        lane = lax.iota(jnp.int32, 16)
        lane_f = lane.astype(jnp.float32)
        negv = jnp.full((16,), NEG, jnp.float32)
        bigf = jnp.float32(1.0e9)

        def start_dma(qi, h):
            pltpu.async_copy(
                scores_hbm.at[qi, pl.ds(h * half, half)], sbuf.at[h], sem0)
            for j in range(nmv):
                pltpu.async_copy(
                    cmax_hbm.at[h * nmv + j, qi], cbuf.at[h, j], sem0)

        def wait_dma(h):
            pltpu.make_async_copy(
                scores_hbm.at[q0, pl.ds(0, half)], sbuf.at[h], sem0).wait()
            for j in range(nmv):
                pltpu.make_async_copy(
                    cmax_hbm.at[0, q0], cbuf.at[h, j], sem0).wait()

        start_dma(q0, 0)

        def elem_trip(vecs, col0f, st):
            m = list(st[:8])
            bv0, bv1, bi0, bi1, tauv = st[8:]
            wv = [jnp.where(m[k] > 0.0, vecs[k], NEG) for k in range(8)]
            mx = wv[0]
            for k in range(1, 8):
                mx = jnp.maximum(mx, wv[k])
            gmax = _bmax(mx)
            posv = jnp.full((16,), bigf, jnp.float32)
            for k in range(8):
                pk = col0f + (k * 16.0) + lane_f
                posv = jnp.minimum(posv,
                                   jnp.where(wv[k] == gmax, pk, bigf))
            pos = _bmin(posv)
            for k in range(8):
                pk = col0f + (k * 16.0) + lane_f
                m[k] = jnp.where(pk == pos, 0.0, m[k])
            upd = gmax > tauv
            ism0 = bv0 == tauv
            ism1 = bv1 == tauv
            ev = _bmax(jnp.maximum(jnp.where(ism0, bi0, -1.0),
                                   jnp.where(ism1, bi1, -1.0)))
            r0 = ism0 & (bi0 == ev) & upd
            r1 = ism1 & (bi1 == ev) & upd
            bv0 = jnp.where(r0, gmax, bv0)
            bv1 = jnp.where(r1, gmax, bv1)
            bi0 = jnp.where(r0, pos, bi0)
            bi1 = jnp.where(r1, pos, bi1)
            tauv = _bmin(jnp.minimum(bv0, bv1))
            return tuple(m) + (bv0, bv1, bi0, bi1, tauv)

        def make_chunk_body(h, mv):
            def chunk_body(ci, st):
                cm = st[0]
                bv0, bv1, bi0, bi1, tauv = st[1:]
                clane = _bmin(jnp.where(cm > 0.0, lane_f, 99.0))
                cm = jnp.where(lane_f == clane, 0.0, cm)
                cl_i = _scal(clane).astype(jnp.int32)
                off = (mv * 16 + cl_i) * CHUNK
                vecs = [sbuf[h, pl.ds(off + k * 16, 16)] for k in range(8)]
                msk = [jnp.where(vecs[k] > tauv, 1.0, 0.0) for k in range(8)]
                csum = msk[0]
                for k in range(1, 8):
                    csum = csum + msk[k]
                ecnt = _scal(_bsum(csum)).astype(jnp.int32)
                col0f = (h * half + off).astype(jnp.float32)
                st2 = lax.fori_loop(
                    0, ecnt, lambda ei, s2: elem_trip(vecs, col0f, s2),
                    tuple(msk) + (bv0, bv1, bi0, bi1, tauv))
                return (cm,) + st2[8:]

            return chunk_body

        def make_mv_body(h):
            def mv_body(mv, st):
                tauv = st[4]
                mvv = cbuf[h, mv, pl.ds(0, 16)]
                cmf = jnp.where(mvv > tauv, 1.0, 0.0)
                cnt = _scal(_bsum(cmf)).astype(jnp.int32)
                st2 = lax.fori_loop(0, cnt, make_chunk_body(h, mv),
                                    (cmf,) + st)
                return st2[1:]

            return mv_body

        def query_body(qs, _):
            qi = q0 + qs
            st = (negv, negv, lane_f, lane_f + 16.0, negv)
            for h in (0, 1):
                @pl.when(qs * 2 + h + 1 < qpw * 2)
                def _start_next(h=h, qi=qi):
                    nh = 1 - h
                    start_dma(qi + (1 if h == 1 else 0), nh)

                wait_dma(h)
                st = lax.fori_loop(0, nmv, make_mv_body(h), st)

            bv0, bv1, bi0, bi1, _tauv = st
            b0, b1, i0, i1 = bv0, bv1, bi0, bi1
            outv = [jnp.zeros((16,), jnp.float32) for _ in range(2)]
            outi = [jnp.zeros((16,), jnp.float32) for _ in range(2)]
            big = jnp.float32(3.0e38)
            for s in range(K):
                mx = _bmax(jnp.maximum(b0, b1))
                t0 = b0 == mx
                t1 = b1 == mx
                mi = _bmin(jnp.minimum(jnp.where(t0, i0, big),
                                       jnp.where(t1, i1, big)))
                w = lane == (s % 16)
                outv[s // 16] = jnp.where(w, mx, outv[s // 16])
                outi[s // 16] = jnp.where(w, mi, outi[s // 16])
                rm0 = t0 & (i0 == mi)
                rm1 = t1 & (i1 == mi)
                b0 = jnp.where(rm0, NEG, b0)
                b1 = jnp.where(rm1, NEG, b1)
            mx0 = _bmax(outv[0])
            e0 = jnp.exp(outv[0] - mx0)
            e1 = jnp.exp(outv[1] - mx0)
            ssum = _bsum(e0) + _bsum(e1)
            a0 = e0 / ssum
            a1 = e1 / ssum
            ti_v[pl.ds(0, 16)] = outi[0].astype(jnp.int32)
            ti_v[pl.ds(16, 16)] = outi[1].astype(jnp.int32)
            att_v[pl.ds(0, 16)] = a0
            att_v[pl.ds(16, 16)] = a1
            pltpu.async_copy(vals_hbm.at[ti_v], rows_v, sem1).wait()
            acc = [jnp.zeros((16,), jnp.float32) for _ in range(nd)]
            for jj in range(K):
                aj = a0 if jj < 16 else a1
                ascal = _perm(aj, jnp.full((16,), jj % 16, jnp.int32))
                for dd in range(nd):
                    acc[dd] = acc[dd] + ascal * rows_v[jj, pl.ds(dd * 16, 16)]
            for dd in range(nd):
                vacc_v[pl.ds(dd * 16, 16)] = acc[dd]
            pltpu.sync_copy(vacc_v, v_out.at[qi])
            pltpu.sync_copy(ti_v, idx_out.at[qi])
            pltpu.sync_copy(att_v, att_out.at[qi])
            return 0

        lax.fori_loop(0, qpw, query_body, 0)

    return topk_kernel(scores, cmax, vals)


def kernel(q, keys, vals, topk):
    n = keys.shape[0]
    npad = -(-n // (2 * KB)) * (2 * KB)
    keys_p = jnp.pad(keys, ((0, npad - n), (0, 0)))
    scores, cmax = _scores(q, keys_p, npad, n)
    v, idx, att = _topk_combine_sc(scores, cmax, vals)
    return (v, idx, att)
